# Initial kernel scaffold; baseline (speedup 1.0000x reference)
import jax
import jax.numpy as jnp
from jax.experimental import pallas as pl


def kernel(src, src_len, src_mask, table):
    raise NotImplementedError("write your pallas kernel here")


# speedup vs baseline: 5.7520x; 5.7520x over previous
import functools

import jax
import jax.numpy as jnp
from jax import lax
from jax.experimental import pallas as pl
from jax.experimental.pallas import tpu as pltpu
from jax.experimental.pallas import tpu_sc as plsc

VOCAB = 100000
HID = 128
SEQ = 200
BATCH = 1024

NC, NS = 2, 16
NW = NC * NS
ROWS = SEQ * BATCH
RPW = ROWS // NW
CHUNK = 128
NCHUNK = RPW // CHUNK

_mesh = plsc.VectorSubcoreMesh(
    core_axis_name="c", subcore_axis_name="s", num_cores=NC, num_subcores=NS
)


@functools.partial(
    pl.kernel,
    out_type=jax.ShapeDtypeStruct((ROWS, HID), jnp.float32),
    mesh=_mesh,
    scratch_types=[
        pltpu.VMEM((NCHUNK, CHUNK), jnp.int32),
        pltpu.VMEM((CHUNK, HID), jnp.float32),
        pltpu.SemaphoreType.DMA,
    ],
)
def _gather_kernel(idx_hbm, table_hbm, out_hbm, idx_v, rows_v, sem):
    wid = lax.axis_index("s") * NC + lax.axis_index("c")
    base = wid * RPW
    pltpu.sync_copy(idx_hbm.at[wid], idx_v)

    def body(j, carry):
        pltpu.async_copy(table_hbm.at[idx_v.at[j]], rows_v, sem).wait()
        pltpu.sync_copy(rows_v, out_hbm.at[pl.ds(base + j * CHUNK, CHUNK)])
        return carry

    lax.fori_loop(0, NCHUNK, body, 0)


def kernel(src, src_len, src_mask, table):
    idx = jnp.transpose(src).reshape(NW, NCHUNK, CHUNK)
    out = _gather_kernel(idx, table)
    return out.reshape(BATCH, SEQ, HID)


# speedup vs baseline: 7.8867x; 1.3711x over previous
import functools

import jax
import jax.numpy as jnp
from jax import lax
from jax.experimental import pallas as pl
from jax.experimental.pallas import tpu as pltpu
from jax.experimental.pallas import tpu_sc as plsc

VOCAB = 100000
HID = 128
SEQ = 200
BATCH = 1024

NC, NS = 2, 16
NW = NC * NS
ROWS = SEQ * BATCH
RPW = ROWS // NW
CHUNK = 128
NCHUNK = RPW // CHUNK

_mesh = plsc.VectorSubcoreMesh(
    core_axis_name="c", subcore_axis_name="s", num_cores=NC, num_subcores=NS
)


NBUF = 2


@functools.partial(
    pl.kernel,
    out_type=jax.ShapeDtypeStruct((ROWS, HID), jnp.float32),
    mesh=_mesh,
    scratch_types=[
        pltpu.VMEM((NCHUNK, CHUNK), jnp.int32),
        [pltpu.VMEM((CHUNK, HID), jnp.float32) for _ in range(NBUF)],
        [pltpu.SemaphoreType.DMA for _ in range(NBUF)],
        [pltpu.SemaphoreType.DMA for _ in range(NBUF)],
    ],
)
def _gather_kernel(idx_hbm, table_hbm, out_hbm, idx_v, rows, gsem, wsem):
    wid = lax.axis_index("s") * NC + lax.axis_index("c")
    base = wid * RPW
    pltpu.sync_copy(idx_hbm.at[wid], idx_v)

    for b in range(NBUF):
        pltpu.async_copy(table_hbm.at[idx_v.at[b]], rows[b], gsem[b])

    def body(i, carry):
        for b in range(NBUF):
            j = i * NBUF + b
            dst = out_hbm.at[pl.ds(base + j * CHUNK, CHUNK)]
            pltpu.make_async_copy(table_hbm.at[idx_v.at[j]], rows[b], gsem[b]).wait()
            pltpu.async_copy(rows[b], dst, wsem[b])

            @pl.when(j + NBUF < NCHUNK)
            def _():
                pltpu.make_async_copy(rows[b], dst, wsem[b]).wait()
                pltpu.async_copy(table_hbm.at[idx_v.at[j + NBUF]], rows[b], gsem[b])

        return carry

    lax.fori_loop(0, NCHUNK // NBUF, body, 0)

    for b in range(NBUF):
        j = NCHUNK - NBUF + b
        dst = out_hbm.at[pl.ds(base + j * CHUNK, CHUNK)]
        pltpu.make_async_copy(rows[b], dst, wsem[b]).wait()


def kernel(src, src_len, src_mask, table):
    idx = jnp.transpose(src).reshape(NW, NCHUNK, CHUNK)
    out = _gather_kernel(idx, table)
    return out.reshape(BATCH, SEQ, HID)


# speedup vs baseline: 8.0807x; 1.0246x over previous
import functools

import jax
import jax.numpy as jnp
from jax import lax
from jax.experimental import pallas as pl
from jax.experimental.pallas import tpu as pltpu
from jax.experimental.pallas import tpu_sc as plsc

VOCAB = 100000
HID = 128
SEQ = 200
BATCH = 1024

NC, NS = 2, 16
NW = NC * NS
ROWS = SEQ * BATCH
RPW = ROWS // NW
CHUNK = 128
NCHUNK = RPW // CHUNK

_mesh = plsc.VectorSubcoreMesh(
    core_axis_name="c", subcore_axis_name="s", num_cores=NC, num_subcores=NS
)


NBUF = 5


@functools.partial(
    pl.kernel,
    out_type=jax.ShapeDtypeStruct((ROWS, HID), jnp.float32),
    mesh=_mesh,
    scratch_types=[
        pltpu.VMEM((NCHUNK, CHUNK), jnp.int32),
        [pltpu.VMEM((CHUNK, HID), jnp.float32) for _ in range(NBUF)],
        [pltpu.SemaphoreType.DMA for _ in range(NBUF)],
        [pltpu.SemaphoreType.DMA for _ in range(NBUF)],
    ],
)
def _gather_kernel(idx_hbm, table_hbm, out_hbm, idx_v, rows, gsem, wsem):
    wid = lax.axis_index("s") * NC + lax.axis_index("c")
    base = wid * RPW
    pltpu.sync_copy(idx_hbm.at[wid], idx_v)

    for b in range(NBUF):
        pltpu.async_copy(table_hbm.at[idx_v.at[b]], rows[b], gsem[b])

    def body(i, carry):
        for b in range(NBUF):
            j = i * NBUF + b
            dst = out_hbm.at[pl.ds(base + j * CHUNK, CHUNK)]
            pltpu.make_async_copy(table_hbm.at[idx_v.at[j]], rows[b], gsem[b]).wait()
            pltpu.async_copy(rows[b], dst, wsem[b])

            @pl.when(j + NBUF < NCHUNK)
            def _():
                pltpu.make_async_copy(rows[b], dst, wsem[b]).wait()
                pltpu.async_copy(table_hbm.at[idx_v.at[j + NBUF]], rows[b], gsem[b])

        return carry

    lax.fori_loop(0, NCHUNK // NBUF, body, 0)

    for b in range(NBUF):
        j = NCHUNK - NBUF + b
        dst = out_hbm.at[pl.ds(base + j * CHUNK, CHUNK)]
        pltpu.make_async_copy(rows[b], dst, wsem[b]).wait()


def kernel(src, src_len, src_mask, table):
    idx = jnp.transpose(src).reshape(NW, NCHUNK, CHUNK)
    out = _gather_kernel(idx, table)
    return out.reshape(BATCH, SEQ, HID)
